# Initial kernel scaffold; baseline (speedup 1.0000x reference)
import jax
import jax.numpy as jnp
from jax.experimental import pallas as pl


def kernel(x, edge_index, W1, a_src1, a_dst1, b1, W2, a_src2, a_dst2, b2):
    raise NotImplementedError("write your pallas kernel here")


# speedup vs baseline: 33.8783x; 33.8783x over previous
import functools

import jax
import jax.numpy as jnp
from jax import lax
from jax.experimental import pallas as pl
from jax.experimental.pallas import tpu as pltpu
from jax.experimental.pallas import tpu_sc as plsc

NN = 10000
NE = 160000
FIN = 256
HEADS = 8
HID = 8
NCLS = 40

NPAD = 10240
EPAD = 163840
CHUNK = 128
NC, NS = 2, 16
NW = NC * NS
E_PER_W = EPAD // NW
CHUNKS_PER_W = E_PER_W // CHUNK
ROWS_PER_TILE = NPAD // NS

BM = 1024


def _vg(x, idx):
    dnums = lax.GatherDimensionNumbers(
        offset_dims=(), collapsed_slice_dims=(0,), start_index_map=(0,))
    return lax.gather(x, idx[:, None], dnums, (1,),
                      mode=lax.GatherScatterMode.PROMISE_IN_BOUNDS)


def _stage_a_body(x_ref, w1e_ref, w1d_ref, t1_ref, d1_ref):
    x = x_ref[...]
    t = jnp.dot(x, w1e_ref[...], preferred_element_type=jnp.float32)
    col = lax.broadcasted_iota(jnp.int32, (BM, 80), 1)
    ones_cols = jnp.where((col >= 64) & (col < 72), 1.0, 0.0)
    t1_ref[...] = t + ones_cols
    d1_ref[...] = jnp.dot(x, w1d_ref[...], preferred_element_type=jnp.float32)


def _stage_a(x_pad, w1e, w1d):
    return pl.pallas_call(
        _stage_a_body,
        grid=(NPAD // BM,),
        in_specs=[
            pl.BlockSpec((BM, FIN), lambda i: (i, 0)),
            pl.BlockSpec((FIN, 80), lambda i: (0, 0)),
            pl.BlockSpec((FIN, 16), lambda i: (0, 0)),
        ],
        out_specs=[
            pl.BlockSpec((BM, 80), lambda i: (i, 0)),
            pl.BlockSpec((BM, 16), lambda i: (i, 0)),
        ],
        out_shape=[
            jax.ShapeDtypeStruct((NPAD, 80), jnp.float32),
            jax.ShapeDtypeStruct((NPAD, 16), jnp.float32),
        ],
    )(x_pad, w1e, w1d)


def _make_sc_layer(rw, a_off, build_msg):
    mesh = plsc.VectorSubcoreMesh(core_axis_name="c", subcore_axis_name="s",
                                  num_cores=NC, num_subcores=NS)

    @functools.partial(
        pl.kernel,
        out_type=jax.ShapeDtypeStruct((NC * NPAD, rw), jnp.float32),
        mesh=mesh,
        scratch_types=[
            pltpu.VMEM((CHUNK,), jnp.int32),
            pltpu.VMEM((CHUNK,), jnp.int32),
            pltpu.VMEM((CHUNK, rw), jnp.float32),
            pltpu.VMEM((CHUNK, 16), jnp.float32),
            pltpu.VMEM((CHUNK, rw), jnp.float32),
            pltpu.VMEM_SHARED((NPAD, rw), jnp.float32),
            pltpu.SemaphoreType.DMA,
            pltpu.SemaphoreType.DMA,
        ],
        compiler_params=pltpu.CompilerParams(use_tc_tiling_on_sc=False, needs_layout_passes=False),
    )
    def sc_layer(t_hbm, d_hbm, src_hbm, dst_hbm, zeros_hbm, out_hbm,
                 srcv, dstv, rowsS, rowsD, msg, acc, sem1, sem2):
        c = lax.axis_index("c")
        s = lax.axis_index("s")
        wid = c * NS + s
        r0 = s * ROWS_PER_TILE
        pltpu.sync_copy(zeros_hbm.at[pl.ds(r0, ROWS_PER_TILE)],
                        acc.at[pl.ds(r0, ROWS_PER_TILE)])
        plsc.subcore_barrier()

        base = wid * E_PER_W

        def chunk_body(k, carry):
            off = base + k * CHUNK
            pltpu.sync_copy(src_hbm.at[pl.ds(off, CHUNK)], srcv)
            pltpu.sync_copy(dst_hbm.at[pl.ds(off, CHUNK)], dstv)
            cp_s = pltpu.async_copy(t_hbm.at[srcv], rowsS, sem1)
            cp_d = pltpu.async_copy(d_hbm.at[dstv], rowsD, sem2)
            cp_s.wait()
            cp_d.wait()

            def edge_body(e, c2):
                a = rowsS[e, pl.ds(a_off, 16)]
                b = rowsD[e, :]
                t = a + b
                w = jnp.exp(jnp.maximum(t, 0.2 * t))
                build_msg(e, rowsS, msg, a, w)
                return c2

            lax.fori_loop(0, CHUNK, edge_body, 0, unroll=4)
            pltpu.sync_copy(msg, acc.at[dstv], add=True)
            return carry

        lax.fori_loop(0, CHUNKS_PER_W, chunk_body, 0)
        plsc.subcore_barrier()
        pltpu.sync_copy(acc.at[pl.ds(r0, ROWS_PER_TILE)],
                        out_hbm.at[pl.ds(c * NPAD + r0, ROWS_PER_TILE)])

    return sc_layer


def _build_msg1(e, rowsS, msg, a, w):
    it = lax.iota(jnp.int32, 16)
    lo = it // 8
    hi = it & 7
    mask8 = jnp.where(it < 8, 1.0, 0.0)
    w01 = _vg(w, 8 + lo)
    w23 = _vg(w, 10 + lo)
    w45 = _vg(w, 12 + lo)
    w67 = _vg(w, 14 + lo)
    wt = _vg(w, 8 + hi) * mask8
    msg[e, pl.ds(0, 16)] = rowsS[e, pl.ds(0, 16)] * w01
    msg[e, pl.ds(16, 16)] = rowsS[e, pl.ds(16, 16)] * w23
    msg[e, pl.ds(32, 16)] = rowsS[e, pl.ds(32, 16)] * w45
    msg[e, pl.ds(48, 16)] = rowsS[e, pl.ds(48, 16)] * w67
    msg[e, pl.ds(64, 16)] = a * wt


def _build_msg2(e, rowsS, msg, a, w):
    it = lax.iota(jnp.int32, 16)
    mask9 = jnp.where(it < 9, 1.0, 0.0)
    wb = _vg(w, jnp.broadcast_to(jnp.int32(9), (16,)))
    msg[e, pl.ds(0, 16)] = rowsS[e, pl.ds(0, 16)] * wb
    msg[e, pl.ds(16, 16)] = rowsS[e, pl.ds(16, 16)] * wb
    msg[e, pl.ds(32, 16)] = a * wb * mask9


@functools.lru_cache(maxsize=None)
def _sc_layers():
    return (_make_sc_layer(80, 64, _build_msg1),
            _make_sc_layer(48, 32, _build_msg2))


def _stage_c_body(t1_ref, d1_ref, accA_ref, accB_ref, b1_ref, w2e_ref,
                  w2d_ref, r8_ref, t2_ref, d2_ref):
    t1 = t1_ref[...]
    h1 = t1[:, 0:64]
    as1 = t1[:, 72:80]
    ad1 = d1_ref[...][:, 8:16]
    sc = as1 + ad1
    wself = jnp.exp(jnp.maximum(sc, 0.2 * sc))
    r8 = r8_ref[...]
    wrep = jnp.dot(wself, r8, preferred_element_type=jnp.float32)
    accA = accA_ref[...]
    accB = accB_ref[...]
    num = accA[:, 0:64] + accB[:, 0:64] + wrep * h1
    den = accA[:, 64:72] + accB[:, 64:72] + wself
    denrep = jnp.dot(den, r8, preferred_element_type=jnp.float32)
    out1 = num / (denrep + 1e-16) + b1_ref[...]
    g = jnp.where(out1 > 0, out1, jnp.exp(jnp.minimum(out1, 0.0)) - 1.0)
    t2 = jnp.dot(g, w2e_ref[...], preferred_element_type=jnp.float32)
    col = lax.broadcasted_iota(jnp.int32, (BM, 48), 1)
    t2_ref[...] = t2 + jnp.where(col == 40, 1.0, 0.0)
    d2_ref[...] = jnp.dot(g, w2d_ref[...], preferred_element_type=jnp.float32)


def _stage_c(t1, d1, acc1, b1row, w2e, w2d, r8):
    nblk = NPAD // BM
    return pl.pallas_call(
        _stage_c_body,
        grid=(nblk,),
        in_specs=[
            pl.BlockSpec((BM, 80), lambda i: (i, 0)),
            pl.BlockSpec((BM, 16), lambda i: (i, 0)),
            pl.BlockSpec((BM, 80), lambda i: (i, 0)),
            pl.BlockSpec((BM, 80), lambda i: (i + NPAD // BM, 0)),
            pl.BlockSpec((1, 64), lambda i: (0, 0)),
            pl.BlockSpec((64, 48), lambda i: (0, 0)),
            pl.BlockSpec((64, 16), lambda i: (0, 0)),
            pl.BlockSpec((8, 64), lambda i: (0, 0)),
        ],
        out_specs=[
            pl.BlockSpec((BM, 48), lambda i: (i, 0)),
            pl.BlockSpec((BM, 16), lambda i: (i, 0)),
        ],
        out_shape=[
            jax.ShapeDtypeStruct((NPAD, 48), jnp.float32),
            jax.ShapeDtypeStruct((NPAD, 16), jnp.float32),
        ],
    )(t1, d1, acc1, acc1, b1row, w2e, w2d, r8)


def _stage_e_body(t2_ref, d2_ref, accA_ref, accB_ref, b2_ref, o_ref):
    t2 = t2_ref[...]
    h2 = t2[:, 0:40]
    as2 = t2[:, 41:42]
    ad2 = d2_ref[...][:, 9:10]
    sc = as2 + ad2
    wself = jnp.exp(jnp.maximum(sc, 0.2 * sc))
    accA = accA_ref[...]
    accB = accB_ref[...]
    num = accA[:, 0:40] + accB[:, 0:40] + wself * h2
    den = accA[:, 40:41] + accB[:, 40:41] + wself
    out = num / (den + 1e-16) + b2_ref[...]
    m = jnp.max(out, axis=1, keepdims=True)
    lse = jnp.log(jnp.sum(jnp.exp(out - m), axis=1, keepdims=True))
    o_ref[...] = out - m - lse


def _stage_e(t2, d2, acc2, b2row):
    return pl.pallas_call(
        _stage_e_body,
        grid=(NPAD // BM,),
        in_specs=[
            pl.BlockSpec((BM, 48), lambda i: (i, 0)),
            pl.BlockSpec((BM, 16), lambda i: (i, 0)),
            pl.BlockSpec((BM, 48), lambda i: (i, 0)),
            pl.BlockSpec((BM, 48), lambda i: (i + NPAD // BM, 0)),
            pl.BlockSpec((1, 40), lambda i: (0, 0)),
        ],
        out_specs=pl.BlockSpec((BM, 40), lambda i: (i, 0)),
        out_shape=jax.ShapeDtypeStruct((NPAD, 40), jnp.float32),
    )(t2, d2, acc2, acc2, b2row)


def kernel(x, edge_index, W1, a_src1, a_dst1, b1, W2, a_src2, a_dst2, b2):
    f32 = jnp.float32
    eye8 = jnp.eye(HEADS, dtype=f32)
    a_s = (a_src1[:, :, None] * eye8[:, None, :]).reshape(HEADS * HID, HEADS)
    a_d = (a_dst1[:, :, None] * eye8[:, None, :]).reshape(HEADS * HID, HEADS)
    w1e = jnp.concatenate(
        [W1, jnp.zeros((FIN, 8), f32), W1 @ a_s], axis=1)
    w1d = jnp.concatenate([jnp.zeros((FIN, 8), f32), W1 @ a_d], axis=1)
    w2e = jnp.concatenate(
        [W2, jnp.zeros((64, 1), f32), W2 @ a_src2.T,
         jnp.zeros((64, 6), f32)], axis=1)
    w2d = jnp.concatenate(
        [jnp.zeros((64, 9), f32), W2 @ a_dst2.T,
         jnp.zeros((64, 6), f32)], axis=1)
    r8 = jnp.kron(jnp.eye(HEADS, dtype=f32), jnp.ones((1, HID), f32))

    x_pad = jnp.concatenate(
        [x, jnp.zeros((NPAD - NN, FIN), f32)], axis=0)
    ei = edge_index.astype(jnp.int32)
    pad_idx = jnp.full((EPAD - NE,), NN, jnp.int32)
    src = jnp.concatenate([ei[0], pad_idx])
    dst = jnp.concatenate([ei[1], pad_idx])
    z80 = jnp.zeros((NPAD, 80), f32)
    z48 = jnp.zeros((NPAD, 48), f32)

    sc1, sc2 = _sc_layers()
    t1, d1 = _stage_a(x_pad, w1e, w1d)
    acc1 = sc1(t1, d1, src, dst, z80)
    t2, d2 = _stage_c(t1, d1, acc1, b1.reshape(1, 64), w2e, w2d, r8)
    acc2 = sc2(t2, d2, src, dst, z48)
    out = _stage_e(t2, d2, acc2, b2.reshape(1, 40))
    return out[:NN]


# speedup vs baseline: 46.5853x; 1.3751x over previous
import functools

import jax
import jax.numpy as jnp
from jax import lax
from jax.experimental import pallas as pl
from jax.experimental.pallas import tpu as pltpu
from jax.experimental.pallas import tpu_sc as plsc

NN = 10000
NE = 160000
FIN = 256
HEADS = 8
HID = 8
NCLS = 40

NPAD = 10240
EPAD = 163840
CHUNK = 128
NC, NS = 2, 16
NW = NC * NS
E_PER_W = EPAD // NW
CHUNKS_PER_W = E_PER_W // CHUNK
ROWS_PER_TILE = NPAD // NS

BM = 1024


def _vg(x, idx):
    dnums = lax.GatherDimensionNumbers(
        offset_dims=(), collapsed_slice_dims=(0,), start_index_map=(0,))
    return lax.gather(x, idx[:, None], dnums, (1,),
                      mode=lax.GatherScatterMode.PROMISE_IN_BOUNDS)


def _stage_a_body(x_ref, w1e_ref, w1d_ref, t1_ref, d1_ref):
    x = x_ref[...]
    t = jnp.dot(x, w1e_ref[...], preferred_element_type=jnp.float32)
    col = lax.broadcasted_iota(jnp.int32, (BM, 80), 1)
    ones_cols = jnp.where((col >= 64) & (col < 72), 1.0, 0.0)
    t1_ref[...] = t + ones_cols
    d1_ref[...] = jnp.dot(x, w1d_ref[...], preferred_element_type=jnp.float32)


def _stage_a(x_pad, w1e, w1d):
    return pl.pallas_call(
        _stage_a_body,
        grid=(NPAD // BM,),
        in_specs=[
            pl.BlockSpec((BM, FIN), lambda i: (i, 0)),
            pl.BlockSpec((FIN, 80), lambda i: (0, 0)),
            pl.BlockSpec((FIN, 16), lambda i: (0, 0)),
        ],
        out_specs=[
            pl.BlockSpec((BM, 80), lambda i: (i, 0)),
            pl.BlockSpec((BM, 16), lambda i: (i, 0)),
        ],
        out_shape=[
            jax.ShapeDtypeStruct((NPAD, 80), jnp.float32),
            jax.ShapeDtypeStruct((NPAD, 16), jnp.float32),
        ],
    )(x_pad, w1e, w1d)


def _make_sc_layer(rw, a_off, build_msg):
    mesh = plsc.VectorSubcoreMesh(core_axis_name="c", subcore_axis_name="s",
                                  num_cores=NC, num_subcores=NS)

    @functools.partial(
        pl.kernel,
        out_type=jax.ShapeDtypeStruct((NC * NPAD, rw), jnp.float32),
        mesh=mesh,
        scratch_types=[
            pltpu.VMEM((CHUNK,), jnp.int32),
            pltpu.VMEM((CHUNK,), jnp.int32),
            pltpu.VMEM((CHUNK, rw), jnp.float32),
            pltpu.VMEM((CHUNK, 16), jnp.float32),
            pltpu.VMEM((CHUNK, rw), jnp.float32),
            pltpu.VMEM_SHARED((NPAD, rw), jnp.float32),
            pltpu.SemaphoreType.DMA,
            pltpu.SemaphoreType.DMA,
        ],
        compiler_params=pltpu.CompilerParams(use_tc_tiling_on_sc=False, needs_layout_passes=False),
    )
    def sc_layer(t_hbm, d_hbm, src_hbm, dst_hbm, zeros_hbm, out_hbm,
                 srcv, dstv, rowsS, rowsD, msg, acc, sem1, sem2):
        c = lax.axis_index("c")
        s = lax.axis_index("s")
        wid = c * NS + s
        r0 = s * ROWS_PER_TILE
        pltpu.sync_copy(zeros_hbm.at[pl.ds(r0, ROWS_PER_TILE)],
                        acc.at[pl.ds(r0, ROWS_PER_TILE)])
        plsc.subcore_barrier()

        base = wid * E_PER_W

        def chunk_body(k, carry):
            off = base + k * CHUNK
            pltpu.sync_copy(src_hbm.at[pl.ds(off, CHUNK)], srcv)
            pltpu.sync_copy(dst_hbm.at[pl.ds(off, CHUNK)], dstv)
            cp_s = pltpu.async_copy(t_hbm.at[srcv], rowsS, sem1)
            cp_d = pltpu.async_copy(d_hbm.at[dstv], rowsD, sem2)
            cp_s.wait()
            cp_d.wait()

            @plsc.parallel_loop(0, CHUNK, unroll=8)
            def edge_body(e):
                a = rowsS[e, pl.ds(a_off, 16)]
                b = rowsD[e, :]
                t = a + b
                w = jnp.exp(jnp.maximum(t, 0.2 * t))
                build_msg(e, rowsS, msg, a, w)
            pltpu.sync_copy(msg, acc.at[dstv], add=True)
            return carry

        lax.fori_loop(0, CHUNKS_PER_W, chunk_body, 0)
        plsc.subcore_barrier()
        pltpu.sync_copy(acc.at[pl.ds(r0, ROWS_PER_TILE)],
                        out_hbm.at[pl.ds(c * NPAD + r0, ROWS_PER_TILE)])

    return sc_layer


def _build_msg1(e, rowsS, msg, a, w):
    it = lax.iota(jnp.int32, 16)
    lo = it // 8
    hi = it & 7
    mask8 = jnp.where(it < 8, 1.0, 0.0)
    w01 = _vg(w, 8 + lo)
    w23 = _vg(w, 10 + lo)
    w45 = _vg(w, 12 + lo)
    w67 = _vg(w, 14 + lo)
    wt = _vg(w, 8 + hi) * mask8
    msg[e, pl.ds(0, 16)] = rowsS[e, pl.ds(0, 16)] * w01
    msg[e, pl.ds(16, 16)] = rowsS[e, pl.ds(16, 16)] * w23
    msg[e, pl.ds(32, 16)] = rowsS[e, pl.ds(32, 16)] * w45
    msg[e, pl.ds(48, 16)] = rowsS[e, pl.ds(48, 16)] * w67
    msg[e, pl.ds(64, 16)] = a * wt


def _build_msg2(e, rowsS, msg, a, w):
    it = lax.iota(jnp.int32, 16)
    mask9 = jnp.where(it < 9, 1.0, 0.0)
    wb = _vg(w, jnp.broadcast_to(jnp.int32(9), (16,)))
    msg[e, pl.ds(0, 16)] = rowsS[e, pl.ds(0, 16)] * wb
    msg[e, pl.ds(16, 16)] = rowsS[e, pl.ds(16, 16)] * wb
    msg[e, pl.ds(32, 16)] = a * wb * mask9


@functools.lru_cache(maxsize=None)
def _sc_layers():
    return (_make_sc_layer(80, 64, _build_msg1),
            _make_sc_layer(48, 32, _build_msg2))


def _stage_c_body(t1_ref, d1_ref, accA_ref, accB_ref, b1_ref, w2e_ref,
                  w2d_ref, r8_ref, t2_ref, d2_ref):
    t1 = t1_ref[...]
    h1 = t1[:, 0:64]
    as1 = t1[:, 72:80]
    ad1 = d1_ref[...][:, 8:16]
    sc = as1 + ad1
    wself = jnp.exp(jnp.maximum(sc, 0.2 * sc))
    r8 = r8_ref[...]
    wrep = jnp.dot(wself, r8, preferred_element_type=jnp.float32)
    accA = accA_ref[...]
    accB = accB_ref[...]
    num = accA[:, 0:64] + accB[:, 0:64] + wrep * h1
    den = accA[:, 64:72] + accB[:, 64:72] + wself
    denrep = jnp.dot(den, r8, preferred_element_type=jnp.float32)
    out1 = num / (denrep + 1e-16) + b1_ref[...]
    g = jnp.where(out1 > 0, out1, jnp.exp(jnp.minimum(out1, 0.0)) - 1.0)
    t2 = jnp.dot(g, w2e_ref[...], preferred_element_type=jnp.float32)
    col = lax.broadcasted_iota(jnp.int32, (BM, 48), 1)
    t2_ref[...] = t2 + jnp.where(col == 40, 1.0, 0.0)
    d2_ref[...] = jnp.dot(g, w2d_ref[...], preferred_element_type=jnp.float32)


def _stage_c(t1, d1, acc1, b1row, w2e, w2d, r8):
    nblk = NPAD // BM
    return pl.pallas_call(
        _stage_c_body,
        grid=(nblk,),
        in_specs=[
            pl.BlockSpec((BM, 80), lambda i: (i, 0)),
            pl.BlockSpec((BM, 16), lambda i: (i, 0)),
            pl.BlockSpec((BM, 80), lambda i: (i, 0)),
            pl.BlockSpec((BM, 80), lambda i: (i + NPAD // BM, 0)),
            pl.BlockSpec((1, 64), lambda i: (0, 0)),
            pl.BlockSpec((64, 48), lambda i: (0, 0)),
            pl.BlockSpec((64, 16), lambda i: (0, 0)),
            pl.BlockSpec((8, 64), lambda i: (0, 0)),
        ],
        out_specs=[
            pl.BlockSpec((BM, 48), lambda i: (i, 0)),
            pl.BlockSpec((BM, 16), lambda i: (i, 0)),
        ],
        out_shape=[
            jax.ShapeDtypeStruct((NPAD, 48), jnp.float32),
            jax.ShapeDtypeStruct((NPAD, 16), jnp.float32),
        ],
    )(t1, d1, acc1, acc1, b1row, w2e, w2d, r8)


def _stage_e_body(t2_ref, d2_ref, accA_ref, accB_ref, b2_ref, o_ref):
    t2 = t2_ref[...]
    h2 = t2[:, 0:40]
    as2 = t2[:, 41:42]
    ad2 = d2_ref[...][:, 9:10]
    sc = as2 + ad2
    wself = jnp.exp(jnp.maximum(sc, 0.2 * sc))
    accA = accA_ref[...]
    accB = accB_ref[...]
    num = accA[:, 0:40] + accB[:, 0:40] + wself * h2
    den = accA[:, 40:41] + accB[:, 40:41] + wself
    out = num / (den + 1e-16) + b2_ref[...]
    m = jnp.max(out, axis=1, keepdims=True)
    lse = jnp.log(jnp.sum(jnp.exp(out - m), axis=1, keepdims=True))
    o_ref[...] = out - m - lse


def _stage_e(t2, d2, acc2, b2row):
    return pl.pallas_call(
        _stage_e_body,
        grid=(NPAD // BM,),
        in_specs=[
            pl.BlockSpec((BM, 48), lambda i: (i, 0)),
            pl.BlockSpec((BM, 16), lambda i: (i, 0)),
            pl.BlockSpec((BM, 48), lambda i: (i, 0)),
            pl.BlockSpec((BM, 48), lambda i: (i + NPAD // BM, 0)),
            pl.BlockSpec((1, 40), lambda i: (0, 0)),
        ],
        out_specs=pl.BlockSpec((BM, 40), lambda i: (i, 0)),
        out_shape=jax.ShapeDtypeStruct((NPAD, 40), jnp.float32),
    )(t2, d2, acc2, acc2, b2row)


def kernel(x, edge_index, W1, a_src1, a_dst1, b1, W2, a_src2, a_dst2, b2):
    f32 = jnp.float32
    eye8 = jnp.eye(HEADS, dtype=f32)
    a_s = (a_src1[:, :, None] * eye8[:, None, :]).reshape(HEADS * HID, HEADS)
    a_d = (a_dst1[:, :, None] * eye8[:, None, :]).reshape(HEADS * HID, HEADS)
    w1e = jnp.concatenate(
        [W1, jnp.zeros((FIN, 8), f32), W1 @ a_s], axis=1)
    w1d = jnp.concatenate([jnp.zeros((FIN, 8), f32), W1 @ a_d], axis=1)
    w2e = jnp.concatenate(
        [W2, jnp.zeros((64, 1), f32), W2 @ a_src2.T,
         jnp.zeros((64, 6), f32)], axis=1)
    w2d = jnp.concatenate(
        [jnp.zeros((64, 9), f32), W2 @ a_dst2.T,
         jnp.zeros((64, 6), f32)], axis=1)
    r8 = jnp.kron(jnp.eye(HEADS, dtype=f32), jnp.ones((1, HID), f32))

    x_pad = jnp.concatenate(
        [x, jnp.zeros((NPAD - NN, FIN), f32)], axis=0)
    ei = edge_index.astype(jnp.int32)
    pad_idx = jnp.full((EPAD - NE,), NN, jnp.int32)
    src = jnp.concatenate([ei[0], pad_idx])
    dst = jnp.concatenate([ei[1], pad_idx])
    z80 = jnp.zeros((NPAD, 80), f32)
    z48 = jnp.zeros((NPAD, 48), f32)

    sc1, sc2 = _sc_layers()
    t1, d1 = _stage_a(x_pad, w1e, w1d)
    acc1 = sc1(t1, d1, src, dst, z80)
    t2, d2 = _stage_c(t1, d1, acc1, b1.reshape(1, 64), w2e, w2d, r8)
    acc2 = sc2(t2, d2, src, dst, z48)
    out = _stage_e(t2, d2, acc2, b2.reshape(1, 40))
    return out[:NN]


# speedup vs baseline: 61.4725x; 1.3196x over previous
import functools

import jax
import jax.numpy as jnp
from jax import lax
from jax.experimental import pallas as pl
from jax.experimental.pallas import tpu as pltpu
from jax.experimental.pallas import tpu_sc as plsc

NN = 10000
NE = 160000
FIN = 256
HEADS = 8
HID = 8
NCLS = 40

NPAD = 10240
EPAD = 163840
CHUNK = 128
NC, NS = 2, 16
NW = NC * NS
E_PER_W = EPAD // NW
CHUNKS_PER_W = E_PER_W // CHUNK
ROWS_PER_TILE = NPAD // NS

BM = 1024


def _vg(x, idx):
    dnums = lax.GatherDimensionNumbers(
        offset_dims=(), collapsed_slice_dims=(0,), start_index_map=(0,))
    return lax.gather(x, idx[:, None], dnums, (1,),
                      mode=lax.GatherScatterMode.PROMISE_IN_BOUNDS)


def _stage_a_body(x_ref, w1e_ref, w1d_ref, t1_ref, d1_ref):
    x = x_ref[...]
    t = jnp.dot(x, w1e_ref[...], preferred_element_type=jnp.float32)
    col = lax.broadcasted_iota(jnp.int32, (BM, 80), 1)
    ones_cols = jnp.where((col >= 64) & (col < 72), 1.0, 0.0)
    t1_ref[...] = t + ones_cols
    d1_ref[...] = jnp.dot(x, w1d_ref[...], preferred_element_type=jnp.float32)


def _stage_a(x_pad, w1e, w1d):
    return pl.pallas_call(
        _stage_a_body,
        grid=(NPAD // BM,),
        in_specs=[
            pl.BlockSpec((BM, FIN), lambda i: (i, 0)),
            pl.BlockSpec((FIN, 80), lambda i: (0, 0)),
            pl.BlockSpec((FIN, 16), lambda i: (0, 0)),
        ],
        out_specs=[
            pl.BlockSpec((BM, 80), lambda i: (i, 0)),
            pl.BlockSpec((BM, 16), lambda i: (i, 0)),
        ],
        out_shape=[
            jax.ShapeDtypeStruct((NPAD, 80), jnp.float32),
            jax.ShapeDtypeStruct((NPAD, 16), jnp.float32),
        ],
    )(x_pad, w1e, w1d)


def _make_sc_layer(rw, a_off, build_msg):
    mesh = plsc.VectorSubcoreMesh(core_axis_name="c", subcore_axis_name="s",
                                  num_cores=NC, num_subcores=NS)

    @functools.partial(
        pl.kernel,
        out_type=jax.ShapeDtypeStruct((NC * NPAD, rw), jnp.float32),
        mesh=mesh,
        scratch_types=[
            pltpu.VMEM((CHUNK,), jnp.int32),
            pltpu.VMEM((CHUNK,), jnp.int32),
            pltpu.VMEM((CHUNK,), jnp.int32),
            pltpu.VMEM((CHUNK,), jnp.int32),
            pltpu.VMEM((CHUNK,), jnp.int32),
            pltpu.VMEM((CHUNK,), jnp.int32),
            pltpu.VMEM((CHUNK, rw), jnp.float32),
            pltpu.VMEM((CHUNK, rw), jnp.float32),
            pltpu.VMEM((CHUNK, 16), jnp.float32),
            pltpu.VMEM((CHUNK, 16), jnp.float32),
            pltpu.VMEM((CHUNK, rw), jnp.float32),
            pltpu.VMEM((CHUNK, rw), jnp.float32),
            pltpu.VMEM_SHARED((NPAD, rw), jnp.float32),
            pltpu.SemaphoreType.DMA,
            pltpu.SemaphoreType.DMA,
            pltpu.SemaphoreType.DMA,
            pltpu.SemaphoreType.DMA,
        ],
        compiler_params=pltpu.CompilerParams(use_tc_tiling_on_sc=False, needs_layout_passes=False),
    )
    def sc_layer(t_hbm, d_hbm, src_hbm, dst_hbm, zeros_hbm, out_hbm,
                 srcv0, dstv0, srcv1, dstv1, sdst0, sdst1,
                 rowsS0, rowsS1, rowsD0, rowsD1, msg0, msg1, acc,
                 gsem0, gsem1, ssem0, ssem1):
        c = lax.axis_index("c")
        s = lax.axis_index("s")
        wid = c * NS + s
        r0 = s * ROWS_PER_TILE
        pltpu.sync_copy(zeros_hbm.at[pl.ds(r0, ROWS_PER_TILE)],
                        acc.at[pl.ds(r0, ROWS_PER_TILE)])
        plsc.subcore_barrier()

        base = wid * E_PER_W
        bufs = (
            (srcv0, dstv0, sdst0, rowsS0, rowsD0, msg0, gsem0, ssem0),
            (srcv1, dstv1, sdst1, rowsS1, rowsD1, msg1, gsem1, ssem1),
        )

        def start_gather(k, b):
            srcv, dstv, _, rowsS, rowsD, _, gsem, _ = b
            off = base + jnp.minimum(k, CHUNKS_PER_W - 1) * CHUNK
            pltpu.sync_copy(src_hbm.at[pl.ds(off, CHUNK)], srcv)
            pltpu.sync_copy(dst_hbm.at[pl.ds(off, CHUNK)], dstv)
            pltpu.async_copy(t_hbm.at[srcv], rowsS, gsem)
            pltpu.async_copy(d_hbm.at[dstv], rowsD, gsem)

        def wait_gather(b):
            srcv, dstv, _, rowsS, rowsD, _, gsem, _ = b
            pltpu.make_async_copy(t_hbm.at[srcv], rowsS, gsem).wait()
            pltpu.make_async_copy(d_hbm.at[dstv], rowsD, gsem).wait()

        def compute(b):
            _, _, _, rowsS, rowsD, msg, _, _ = b

            @plsc.parallel_loop(0, CHUNK, unroll=8)
            def edge_body(e):
                a = rowsS[e, pl.ds(a_off, 16)]
                bb = rowsD[e, :]
                t = a + bb
                w = jnp.exp(jnp.maximum(t, 0.2 * t))
                build_msg(e, rowsS, msg, a, w)

        def start_scatter(k, b):
            _, _, sdst, _, _, msg, _, ssem = b
            off = base + k * CHUNK
            pltpu.sync_copy(dst_hbm.at[pl.ds(off, CHUNK)], sdst)
            pltpu.async_copy(msg, acc.at[sdst], ssem, add=True)

        def wait_scatter(b):
            _, _, sdst, _, _, msg, _, ssem = b
            pltpu.make_async_copy(msg, acc.at[sdst], ssem).wait()

        start_gather(0, bufs[0])

        def pair_body(j, carry):
            k0 = 2 * j
            start_gather(k0 + 1, bufs[1])
            wait_gather(bufs[0])

            @pl.when(j > 0)
            def _():
                wait_scatter(bufs[0])

            compute(bufs[0])
            start_scatter(k0, bufs[0])
            start_gather(k0 + 2, bufs[0])
            wait_gather(bufs[1])

            @pl.when(j > 0)
            def _():
                wait_scatter(bufs[1])

            compute(bufs[1])
            start_scatter(k0 + 1, bufs[1])
            return carry

        lax.fori_loop(0, CHUNKS_PER_W // 2, pair_body, 0)
        wait_gather(bufs[0])
        wait_scatter(bufs[0])
        wait_scatter(bufs[1])
        plsc.subcore_barrier()
        pltpu.sync_copy(acc.at[pl.ds(r0, ROWS_PER_TILE)],
                        out_hbm.at[pl.ds(c * NPAD + r0, ROWS_PER_TILE)])

    return sc_layer


def _build_msg1(e, rowsS, msg, a, w):
    it = lax.iota(jnp.int32, 16)
    lo = it // 8
    hi = it & 7
    mask8 = jnp.where(it < 8, 1.0, 0.0)
    w01 = _vg(w, 8 + lo)
    w23 = _vg(w, 10 + lo)
    w45 = _vg(w, 12 + lo)
    w67 = _vg(w, 14 + lo)
    wt = _vg(w, 8 + hi) * mask8
    msg[e, pl.ds(0, 16)] = rowsS[e, pl.ds(0, 16)] * w01
    msg[e, pl.ds(16, 16)] = rowsS[e, pl.ds(16, 16)] * w23
    msg[e, pl.ds(32, 16)] = rowsS[e, pl.ds(32, 16)] * w45
    msg[e, pl.ds(48, 16)] = rowsS[e, pl.ds(48, 16)] * w67
    msg[e, pl.ds(64, 16)] = a * wt


def _build_msg2(e, rowsS, msg, a, w):
    it = lax.iota(jnp.int32, 16)
    mask9 = jnp.where(it < 9, 1.0, 0.0)
    wb = _vg(w, jnp.broadcast_to(jnp.int32(9), (16,)))
    msg[e, pl.ds(0, 16)] = rowsS[e, pl.ds(0, 16)] * wb
    msg[e, pl.ds(16, 16)] = rowsS[e, pl.ds(16, 16)] * wb
    msg[e, pl.ds(32, 16)] = a * wb * mask9


@functools.lru_cache(maxsize=None)
def _sc_layers():
    return (_make_sc_layer(80, 64, _build_msg1),
            _make_sc_layer(48, 32, _build_msg2))


def _stage_c_body(t1_ref, d1_ref, accA_ref, accB_ref, b1_ref, w2e_ref,
                  w2d_ref, r8_ref, t2_ref, d2_ref):
    t1 = t1_ref[...]
    h1 = t1[:, 0:64]
    as1 = t1[:, 72:80]
    ad1 = d1_ref[...][:, 8:16]
    sc = as1 + ad1
    wself = jnp.exp(jnp.maximum(sc, 0.2 * sc))
    r8 = r8_ref[...]
    wrep = jnp.dot(wself, r8, preferred_element_type=jnp.float32)
    accA = accA_ref[...]
    accB = accB_ref[...]
    num = accA[:, 0:64] + accB[:, 0:64] + wrep * h1
    den = accA[:, 64:72] + accB[:, 64:72] + wself
    denrep = jnp.dot(den, r8, preferred_element_type=jnp.float32)
    out1 = num / (denrep + 1e-16) + b1_ref[...]
    g = jnp.where(out1 > 0, out1, jnp.exp(jnp.minimum(out1, 0.0)) - 1.0)
    t2 = jnp.dot(g, w2e_ref[...], preferred_element_type=jnp.float32)
    col = lax.broadcasted_iota(jnp.int32, (BM, 48), 1)
    t2_ref[...] = t2 + jnp.where(col == 40, 1.0, 0.0)
    d2_ref[...] = jnp.dot(g, w2d_ref[...], preferred_element_type=jnp.float32)


def _stage_c(t1, d1, acc1, b1row, w2e, w2d, r8):
    nblk = NPAD // BM
    return pl.pallas_call(
        _stage_c_body,
        grid=(nblk,),
        in_specs=[
            pl.BlockSpec((BM, 80), lambda i: (i, 0)),
            pl.BlockSpec((BM, 16), lambda i: (i, 0)),
            pl.BlockSpec((BM, 80), lambda i: (i, 0)),
            pl.BlockSpec((BM, 80), lambda i: (i + NPAD // BM, 0)),
            pl.BlockSpec((1, 64), lambda i: (0, 0)),
            pl.BlockSpec((64, 48), lambda i: (0, 0)),
            pl.BlockSpec((64, 16), lambda i: (0, 0)),
            pl.BlockSpec((8, 64), lambda i: (0, 0)),
        ],
        out_specs=[
            pl.BlockSpec((BM, 48), lambda i: (i, 0)),
            pl.BlockSpec((BM, 16), lambda i: (i, 0)),
        ],
        out_shape=[
            jax.ShapeDtypeStruct((NPAD, 48), jnp.float32),
            jax.ShapeDtypeStruct((NPAD, 16), jnp.float32),
        ],
    )(t1, d1, acc1, acc1, b1row, w2e, w2d, r8)


def _stage_e_body(t2_ref, d2_ref, accA_ref, accB_ref, b2_ref, o_ref):
    t2 = t2_ref[...]
    h2 = t2[:, 0:40]
    as2 = t2[:, 41:42]
    ad2 = d2_ref[...][:, 9:10]
    sc = as2 + ad2
    wself = jnp.exp(jnp.maximum(sc, 0.2 * sc))
    accA = accA_ref[...]
    accB = accB_ref[...]
    num = accA[:, 0:40] + accB[:, 0:40] + wself * h2
    den = accA[:, 40:41] + accB[:, 40:41] + wself
    out = num / (den + 1e-16) + b2_ref[...]
    m = jnp.max(out, axis=1, keepdims=True)
    lse = jnp.log(jnp.sum(jnp.exp(out - m), axis=1, keepdims=True))
    o_ref[...] = out - m - lse


def _stage_e(t2, d2, acc2, b2row):
    return pl.pallas_call(
        _stage_e_body,
        grid=(NPAD // BM,),
        in_specs=[
            pl.BlockSpec((BM, 48), lambda i: (i, 0)),
            pl.BlockSpec((BM, 16), lambda i: (i, 0)),
            pl.BlockSpec((BM, 48), lambda i: (i, 0)),
            pl.BlockSpec((BM, 48), lambda i: (i + NPAD // BM, 0)),
            pl.BlockSpec((1, 40), lambda i: (0, 0)),
        ],
        out_specs=pl.BlockSpec((BM, 40), lambda i: (i, 0)),
        out_shape=jax.ShapeDtypeStruct((NPAD, 40), jnp.float32),
    )(t2, d2, acc2, acc2, b2row)


def kernel(x, edge_index, W1, a_src1, a_dst1, b1, W2, a_src2, a_dst2, b2):
    f32 = jnp.float32
    eye8 = jnp.eye(HEADS, dtype=f32)
    a_s = (a_src1[:, :, None] * eye8[:, None, :]).reshape(HEADS * HID, HEADS)
    a_d = (a_dst1[:, :, None] * eye8[:, None, :]).reshape(HEADS * HID, HEADS)
    w1e = jnp.concatenate(
        [W1, jnp.zeros((FIN, 8), f32), W1 @ a_s], axis=1)
    w1d = jnp.concatenate([jnp.zeros((FIN, 8), f32), W1 @ a_d], axis=1)
    w2e = jnp.concatenate(
        [W2, jnp.zeros((64, 1), f32), W2 @ a_src2.T,
         jnp.zeros((64, 6), f32)], axis=1)
    w2d = jnp.concatenate(
        [jnp.zeros((64, 9), f32), W2 @ a_dst2.T,
         jnp.zeros((64, 6), f32)], axis=1)
    r8 = jnp.kron(jnp.eye(HEADS, dtype=f32), jnp.ones((1, HID), f32))

    x_pad = jnp.concatenate(
        [x, jnp.zeros((NPAD - NN, FIN), f32)], axis=0)
    ei = edge_index.astype(jnp.int32)
    pad_idx = jnp.full((EPAD - NE,), NN, jnp.int32)
    src = jnp.concatenate([ei[0], pad_idx])
    dst = jnp.concatenate([ei[1], pad_idx])
    z80 = jnp.zeros((NPAD, 80), f32)
    z48 = jnp.zeros((NPAD, 48), f32)

    sc1, sc2 = _sc_layers()
    t1, d1 = _stage_a(x_pad, w1e, w1d)
    acc1 = sc1(t1, d1, src, dst, z80)
    t2, d2 = _stage_c(t1, d1, acc1, b1.reshape(1, 64), w2e, w2d, r8)
    acc2 = sc2(t2, d2, src, dst, z48)
    out = _stage_e(t2, d2, acc2, b2.reshape(1, 40))
    return out[:NN]


# speedup vs baseline: 63.7899x; 1.0377x over previous
import functools

import jax
import jax.numpy as jnp
from jax import lax
from jax.experimental import pallas as pl
from jax.experimental.pallas import tpu as pltpu
from jax.experimental.pallas import tpu_sc as plsc

NN = 10000
NE = 160000
FIN = 256
HEADS = 8
HID = 8
NCLS = 40

NPAD = 10240
EPAD = 163840
CHUNK = 128
NC, NS = 2, 16
NW = NC * NS
E_PER_W = EPAD // NW
CHUNKS_PER_W = E_PER_W // CHUNK
ROWS_PER_TILE = NPAD // NS

BM = 1024


def _vg(x, idx):
    dnums = lax.GatherDimensionNumbers(
        offset_dims=(), collapsed_slice_dims=(0,), start_index_map=(0,))
    return lax.gather(x, idx[:, None], dnums, (1,),
                      mode=lax.GatherScatterMode.PROMISE_IN_BOUNDS)


def _stage_a_body(x_ref, w1e_ref, w1d_ref, t1_ref, d1_ref):
    x = x_ref[...]
    t = jnp.dot(x, w1e_ref[...], preferred_element_type=jnp.float32)
    col = lax.broadcasted_iota(jnp.int32, (BM, 80), 1)
    ones_cols = jnp.where((col >= 64) & (col < 72), 1.0, 0.0)
    t1_ref[...] = t + ones_cols
    d1_ref[...] = jnp.dot(x, w1d_ref[...], preferred_element_type=jnp.float32)


def _stage_a(x_pad, w1e, w1d):
    return pl.pallas_call(
        _stage_a_body,
        grid=(NPAD // BM,),
        in_specs=[
            pl.BlockSpec((BM, FIN), lambda i: (i, 0)),
            pl.BlockSpec((FIN, 80), lambda i: (0, 0)),
            pl.BlockSpec((FIN, 16), lambda i: (0, 0)),
        ],
        out_specs=[
            pl.BlockSpec((BM, 80), lambda i: (i, 0)),
            pl.BlockSpec((BM, 16), lambda i: (i, 0)),
        ],
        out_shape=[
            jax.ShapeDtypeStruct((NPAD, 80), jnp.float32),
            jax.ShapeDtypeStruct((NPAD, 16), jnp.float32),
        ],
    )(x_pad, w1e, w1d)


def _make_sc_layer(rw, a_off, build_msg):
    mesh = plsc.VectorSubcoreMesh(core_axis_name="c", subcore_axis_name="s",
                                  num_cores=NC, num_subcores=NS)

    @functools.partial(
        pl.kernel,
        out_type=jax.ShapeDtypeStruct((NC * NPAD, rw), jnp.float32),
        mesh=mesh,
        scratch_types=[
            pltpu.VMEM((CHUNKS_PER_W, CHUNK), jnp.int32),
            pltpu.VMEM((CHUNKS_PER_W, CHUNK), jnp.int32),
            pltpu.VMEM((CHUNK, rw), jnp.float32),
            pltpu.VMEM((CHUNK, rw), jnp.float32),
            pltpu.VMEM((CHUNK, 16), jnp.float32),
            pltpu.VMEM((CHUNK, 16), jnp.float32),
            pltpu.VMEM((CHUNK, rw), jnp.float32),
            pltpu.VMEM((CHUNK, rw), jnp.float32),
            pltpu.VMEM_SHARED((NPAD, rw), jnp.float32),
            pltpu.SemaphoreType.DMA,
            pltpu.SemaphoreType.DMA,
            pltpu.SemaphoreType.DMA,
            pltpu.SemaphoreType.DMA,
            pltpu.SemaphoreType.DMA,
        ],
        compiler_params=pltpu.CompilerParams(use_tc_tiling_on_sc=False, needs_layout_passes=False),
    )
    def sc_layer(t_hbm, d_hbm, src_hbm, dst_hbm, zeros_hbm, out_hbm,
                 srcs, dsts, rowsS0, rowsS1, rowsD0, rowsD1, msg0, msg1, acc,
                 isem, gsem0, gsem1, ssem0, ssem1):
        c = lax.axis_index("c")
        s = lax.axis_index("s")
        wid = c * NS + s
        r0 = s * ROWS_PER_TILE
        cp_si = pltpu.async_copy(src_hbm.at[wid], srcs, isem)
        cp_di = pltpu.async_copy(dst_hbm.at[wid], dsts, isem)
        pltpu.sync_copy(zeros_hbm.at[pl.ds(r0, ROWS_PER_TILE)],
                        acc.at[pl.ds(r0, ROWS_PER_TILE)])
        cp_si.wait()
        cp_di.wait()
        plsc.subcore_barrier()

        bufs = (
            (rowsS0, rowsD0, msg0, gsem0, ssem0),
            (rowsS1, rowsD1, msg1, gsem1, ssem1),
        )

        def start_gather(k, b):
            rowsS, rowsD, _, gsem, _ = b
            kc = jnp.minimum(k, CHUNKS_PER_W - 1)
            pltpu.async_copy(t_hbm.at[srcs.at[kc]], rowsS, gsem)
            pltpu.async_copy(d_hbm.at[dsts.at[kc]], rowsD, gsem)

        def wait_gather(k, b):
            rowsS, rowsD, _, gsem, _ = b
            kc = jnp.minimum(k, CHUNKS_PER_W - 1)
            pltpu.make_async_copy(t_hbm.at[srcs.at[kc]], rowsS, gsem).wait()
            pltpu.make_async_copy(d_hbm.at[dsts.at[kc]], rowsD, gsem).wait()

        def compute(b):
            rowsS, rowsD, msg, _, _ = b

            @plsc.parallel_loop(0, CHUNK, unroll=8)
            def edge_body(e):
                a = rowsS[e, pl.ds(a_off, 16)]
                bb = rowsD[e, :]
                t = a + bb
                w = jnp.exp(jnp.maximum(t, 0.2 * t))
                build_msg(e, rowsS, msg, a, w)

        def start_scatter(k, b):
            _, _, msg, _, ssem = b
            pltpu.async_copy(msg, acc.at[dsts.at[k]], ssem, add=True)

        def wait_scatter(k, b):
            _, _, msg, _, ssem = b
            pltpu.make_async_copy(msg, acc.at[dsts.at[k]], ssem).wait()

        start_gather(0, bufs[0])

        def pair_body(j, carry):
            k0 = 2 * j
            start_gather(k0 + 1, bufs[1])
            wait_gather(k0, bufs[0])

            @pl.when(j > 0)
            def _():
                wait_scatter(k0 - 2, bufs[0])

            compute(bufs[0])
            start_scatter(k0, bufs[0])
            start_gather(k0 + 2, bufs[0])
            wait_gather(k0 + 1, bufs[1])

            @pl.when(j > 0)
            def _():
                wait_scatter(k0 - 1, bufs[1])

            compute(bufs[1])
            start_scatter(k0 + 1, bufs[1])
            return carry

        lax.fori_loop(0, CHUNKS_PER_W // 2, pair_body, 0)
        wait_gather(CHUNKS_PER_W, bufs[0])
        wait_scatter(CHUNKS_PER_W - 2, bufs[0])
        wait_scatter(CHUNKS_PER_W - 1, bufs[1])
        plsc.subcore_barrier()
        pltpu.sync_copy(acc.at[pl.ds(r0, ROWS_PER_TILE)],
                        out_hbm.at[pl.ds(c * NPAD + r0, ROWS_PER_TILE)])

    return sc_layer


def _build_msg1(e, rowsS, msg, a, w):
    it = lax.iota(jnp.int32, 16)
    lo = it // 8
    hi = it & 7
    mask8 = jnp.where(it < 8, 1.0, 0.0)
    w01 = _vg(w, 8 + lo)
    w23 = _vg(w, 10 + lo)
    w45 = _vg(w, 12 + lo)
    w67 = _vg(w, 14 + lo)
    wt = _vg(w, 8 + hi) * mask8
    msg[e, pl.ds(0, 16)] = rowsS[e, pl.ds(0, 16)] * w01
    msg[e, pl.ds(16, 16)] = rowsS[e, pl.ds(16, 16)] * w23
    msg[e, pl.ds(32, 16)] = rowsS[e, pl.ds(32, 16)] * w45
    msg[e, pl.ds(48, 16)] = rowsS[e, pl.ds(48, 16)] * w67
    msg[e, pl.ds(64, 16)] = a * wt


def _build_msg2(e, rowsS, msg, a, w):
    it = lax.iota(jnp.int32, 16)
    mask9 = jnp.where(it < 9, 1.0, 0.0)
    wb = _vg(w, jnp.broadcast_to(jnp.int32(9), (16,)))
    msg[e, pl.ds(0, 16)] = rowsS[e, pl.ds(0, 16)] * wb
    msg[e, pl.ds(16, 16)] = rowsS[e, pl.ds(16, 16)] * wb
    msg[e, pl.ds(32, 16)] = a * wb * mask9


@functools.lru_cache(maxsize=None)
def _sc_layers():
    return (_make_sc_layer(80, 64, _build_msg1),
            _make_sc_layer(48, 32, _build_msg2))


def _stage_c_body(t1_ref, d1_ref, accA_ref, accB_ref, b1_ref, w2e_ref,
                  w2d_ref, r8_ref, t2_ref, d2_ref):
    t1 = t1_ref[...]
    h1 = t1[:, 0:64]
    as1 = t1[:, 72:80]
    ad1 = d1_ref[...][:, 8:16]
    sc = as1 + ad1
    wself = jnp.exp(jnp.maximum(sc, 0.2 * sc))
    r8 = r8_ref[...]
    wrep = jnp.dot(wself, r8, preferred_element_type=jnp.float32)
    accA = accA_ref[...]
    accB = accB_ref[...]
    num = accA[:, 0:64] + accB[:, 0:64] + wrep * h1
    den = accA[:, 64:72] + accB[:, 64:72] + wself
    denrep = jnp.dot(den, r8, preferred_element_type=jnp.float32)
    out1 = num / (denrep + 1e-16) + b1_ref[...]
    g = jnp.where(out1 > 0, out1, jnp.exp(jnp.minimum(out1, 0.0)) - 1.0)
    t2 = jnp.dot(g, w2e_ref[...], preferred_element_type=jnp.float32)
    col = lax.broadcasted_iota(jnp.int32, (BM, 48), 1)
    t2_ref[...] = t2 + jnp.where(col == 40, 1.0, 0.0)
    d2_ref[...] = jnp.dot(g, w2d_ref[...], preferred_element_type=jnp.float32)


def _stage_c(t1, d1, acc1, b1row, w2e, w2d, r8):
    nblk = NPAD // BM
    return pl.pallas_call(
        _stage_c_body,
        grid=(nblk,),
        in_specs=[
            pl.BlockSpec((BM, 80), lambda i: (i, 0)),
            pl.BlockSpec((BM, 16), lambda i: (i, 0)),
            pl.BlockSpec((BM, 80), lambda i: (i, 0)),
            pl.BlockSpec((BM, 80), lambda i: (i + NPAD // BM, 0)),
            pl.BlockSpec((1, 64), lambda i: (0, 0)),
            pl.BlockSpec((64, 48), lambda i: (0, 0)),
            pl.BlockSpec((64, 16), lambda i: (0, 0)),
            pl.BlockSpec((8, 64), lambda i: (0, 0)),
        ],
        out_specs=[
            pl.BlockSpec((BM, 48), lambda i: (i, 0)),
            pl.BlockSpec((BM, 16), lambda i: (i, 0)),
        ],
        out_shape=[
            jax.ShapeDtypeStruct((NPAD, 48), jnp.float32),
            jax.ShapeDtypeStruct((NPAD, 16), jnp.float32),
        ],
    )(t1, d1, acc1, acc1, b1row, w2e, w2d, r8)


def _stage_e_body(t2_ref, d2_ref, accA_ref, accB_ref, b2_ref, o_ref):
    t2 = t2_ref[...]
    h2 = t2[:, 0:40]
    as2 = t2[:, 41:42]
    ad2 = d2_ref[...][:, 9:10]
    sc = as2 + ad2
    wself = jnp.exp(jnp.maximum(sc, 0.2 * sc))
    accA = accA_ref[...]
    accB = accB_ref[...]
    num = accA[:, 0:40] + accB[:, 0:40] + wself * h2
    den = accA[:, 40:41] + accB[:, 40:41] + wself
    out = num / (den + 1e-16) + b2_ref[...]
    m = jnp.max(out, axis=1, keepdims=True)
    lse = jnp.log(jnp.sum(jnp.exp(out - m), axis=1, keepdims=True))
    o_ref[...] = out - m - lse


def _stage_e(t2, d2, acc2, b2row):
    return pl.pallas_call(
        _stage_e_body,
        grid=(NPAD // BM,),
        in_specs=[
            pl.BlockSpec((BM, 48), lambda i: (i, 0)),
            pl.BlockSpec((BM, 16), lambda i: (i, 0)),
            pl.BlockSpec((BM, 48), lambda i: (i, 0)),
            pl.BlockSpec((BM, 48), lambda i: (i + NPAD // BM, 0)),
            pl.BlockSpec((1, 40), lambda i: (0, 0)),
        ],
        out_specs=pl.BlockSpec((BM, 40), lambda i: (i, 0)),
        out_shape=jax.ShapeDtypeStruct((NPAD, 40), jnp.float32),
    )(t2, d2, acc2, acc2, b2row)


def kernel(x, edge_index, W1, a_src1, a_dst1, b1, W2, a_src2, a_dst2, b2):
    f32 = jnp.float32
    eye8 = jnp.eye(HEADS, dtype=f32)
    a_s = (a_src1[:, :, None] * eye8[:, None, :]).reshape(HEADS * HID, HEADS)
    a_d = (a_dst1[:, :, None] * eye8[:, None, :]).reshape(HEADS * HID, HEADS)
    w1e = jnp.concatenate(
        [W1, jnp.zeros((FIN, 8), f32), W1 @ a_s], axis=1)
    w1d = jnp.concatenate([jnp.zeros((FIN, 8), f32), W1 @ a_d], axis=1)
    w2e = jnp.concatenate(
        [W2, jnp.zeros((64, 1), f32), W2 @ a_src2.T,
         jnp.zeros((64, 6), f32)], axis=1)
    w2d = jnp.concatenate(
        [jnp.zeros((64, 9), f32), W2 @ a_dst2.T,
         jnp.zeros((64, 6), f32)], axis=1)
    r8 = jnp.kron(jnp.eye(HEADS, dtype=f32), jnp.ones((1, HID), f32))

    x_pad = jnp.concatenate(
        [x, jnp.zeros((NPAD - NN, FIN), f32)], axis=0)
    ei = edge_index.astype(jnp.int32)
    pad_idx = jnp.full((EPAD - NE,), NN, jnp.int32)
    src = jnp.concatenate([ei[0], pad_idx]).reshape(NW, CHUNKS_PER_W, CHUNK)
    dst = jnp.concatenate([ei[1], pad_idx]).reshape(NW, CHUNKS_PER_W, CHUNK)
    z80 = jnp.zeros((NPAD, 80), f32)
    z48 = jnp.zeros((NPAD, 48), f32)

    sc1, sc2 = _sc_layers()
    t1, d1 = _stage_a(x_pad, w1e, w1d)
    acc1 = sc1(t1, d1, src, dst, z80)
    t2, d2 = _stage_c(t1, d1, acc1, b1.reshape(1, 64), w2e, w2d, r8)
    acc2 = sc2(t2, d2, src, dst, z48)
    out = _stage_e(t2, d2, acc2, b2.reshape(1, 40))
    return out[:NN]
